# baseline (speedup 1.0000x reference)
import functools

import jax
import jax.numpy as jnp
from jax import lax
from jax.experimental import pallas as pl
from jax.experimental.pallas import tpu as pltpu
from jax.experimental.pallas import tpu_sc as plsc

N_BITS_K = 1024
HASH_BITS_K = 14
N_HASH_K = 3
BATCH_K = 16384
RAM_K = 2 ** HASH_BITS_K
NW_K = 1024 // 4


_BB = 512


def _addr_body(w_ref, bits_ref, o0, o1, o2):
    b = bits_ref[...].astype(jnp.float32)
    res = jnp.dot(b, w_ref[...], preferred_element_type=jnp.float32)
    resi = res.astype(jnp.int32)
    o0[...] = resi[:, 0:1]
    o1[...] = resi[:, 1:2] + RAM_K
    o2[...] = resi[:, 2:3] + 2 * RAM_K


def _addr_call(positions, bits):
    cols = (N_BITS_K - 1 - positions).reshape(-1)
    hh = jnp.repeat(jnp.arange(N_HASH_K), HASH_BITS_K)
    wts = jnp.tile(2.0 ** jnp.arange(HASH_BITS_K - 1, -1, -1), N_HASH_K)
    w = jnp.zeros((N_BITS_K, 8), jnp.float32).at[cols, hh].add(
        wts.astype(jnp.float32))
    grid = (BATCH_K // _BB,)
    out = jax.ShapeDtypeStruct((BATCH_K, 1), jnp.int32)
    a0, a1, a2 = pl.pallas_call(
        _addr_body,
        grid=grid,
        in_specs=[
            pl.BlockSpec((N_BITS_K, 8), lambda i: (0, 0)),
            pl.BlockSpec((_BB, N_BITS_K), lambda i: (i, 0)),
        ],
        out_specs=[
            pl.BlockSpec((_BB, 1), lambda i: (i, 0)),
            pl.BlockSpec((_BB, 1), lambda i: (i, 0)),
            pl.BlockSpec((_BB, 1), lambda i: (i, 0)),
        ],
        out_shape=[out, out, out],
    )(w, bits)
    return a0.reshape(-1), a1.reshape(-1), a2.reshape(-1)


_BN = 512
_BA = 256


def _tr_body(s_ref, mem_ref, out_ref):
    x = mem_ref[0]
    t = lax.dot_general(
        x, s_ref[...],
        dimension_numbers=(((0,), (1,)), ((), ())),
        preferred_element_type=jnp.float32)
    q = _BN // 4
    acc = t[:, 0:q].astype(jnp.int32)
    acc = acc + (t[:, q:2 * q].astype(jnp.int32) << 8)
    acc = acc + (t[:, 2 * q:3 * q].astype(jnp.int32) << 16)
    acc = acc + (t[:, 3 * q:4 * q].astype(jnp.int32) << 24)
    out_ref[0] = acc


def _pack_mat():
    m = jnp.arange(_BN)[:, None]
    n = jnp.arange(_BN)[None, :]
    k = m % (_BN // 4)
    r = m // (_BN // 4)
    return (n == 4 * k + r).astype(jnp.float32)


def _tr_call(memory):
    grid = (N_HASH_K, N_BITS_K // _BN, RAM_K // _BA)
    smat = _pack_mat()
    return pl.pallas_call(
        _tr_body,
        grid=grid,
        in_specs=[pl.BlockSpec((_BN, _BN), lambda h, nb, ab: (0, 0)),
                  pl.BlockSpec((1, _BN, _BA), lambda h, nb, ab: (h, nb, ab))],
        out_specs=pl.BlockSpec((1, _BA, _BN // 4), lambda h, nb, ab: (h, ab, nb)),
        out_shape=jax.ShapeDtypeStruct((N_HASH_K, RAM_K, NW_K), jnp.int32),
    )(smat, memory)


_NSC = 32
_BPW = BATCH_K // _NSC
_CH = 64
_NCH = _BPW // _CH
_SG = 4
_SR = _CH // _SG


def _sc_body(table_hbm, a0_hbm, a1_hbm, a2_hbm, out_hbm,
             i0, i1, i2, b00, b01, b02, b10, b11, b12, sem_a, sem_b):
    wid = lax.axis_index("s") * 2 + lax.axis_index("c")
    base = wid * _BPW
    pltpu.sync_copy(a0_hbm.at[pl.ds(base, _BPW)], i0)
    pltpu.sync_copy(a1_hbm.at[pl.ds(base, _BPW)], i1)
    pltpu.sync_copy(a2_hbm.at[pl.ds(base, _BPW)], i2)

    set_a = (b00, b01, b02, sem_a)
    set_b = (b10, b11, b12, sem_b)

    def fire(c, bset):
        bb0, bb1, bb2, sm = bset
        sl = pl.ds(c * _CH, _CH)
        pltpu.async_copy(table_hbm.at[i0.at[sl]], bb0.reshape(_CH, NW_K), sm)
        pltpu.async_copy(table_hbm.at[i1.at[sl]], bb1.reshape(_CH, NW_K), sm)
        pltpu.async_copy(table_hbm.at[i2.at[sl]], bb2.reshape(_CH, NW_K), sm)

    def drain(c, bset):
        bb0, bb1, bb2, sm = bset
        sl = pl.ds(c * _CH, _CH)
        pltpu.make_async_copy(table_hbm.at[i0.at[sl]], bb0.reshape(_CH, NW_K), sm).wait()
        pltpu.make_async_copy(table_hbm.at[i1.at[sl]], bb1.reshape(_CH, NW_K), sm).wait()
        pltpu.make_async_copy(table_hbm.at[i2.at[sl]], bb2.reshape(_CH, NW_K), sm).wait()

    def compute_write(c, bset):
        bb0, bb1, bb2, _ = bset

        def maj_body(s, _):
            for r in range(_SR):
                for j in range(NW_K // 16):
                    sl = pl.ds(j * 16, 16)
                    v0 = bb0[s, r, sl]
                    v1 = bb1[s, r, sl]
                    v2 = bb2[s, r, sl]
                    bb0[s, r, sl] = (v0 & v1) | (v2 & (v0 | v1))
            return 0

        lax.fori_loop(0, _SG, maj_body, 0)
        pltpu.sync_copy(
            bb0, out_hbm.at[pl.ds((base + c * _CH) // _SR, _SG)])

    fire(0, set_a)

    def pair_body(t, _):
        c0 = 2 * t
        c1 = 2 * t + 1
        fire(c1, set_b)
        drain(c0, set_a)
        compute_write(c0, set_a)
        fire((c1 + 1) % _NCH, set_a)
        drain(c1, set_b)
        compute_write(c1, set_b)
        return 0

    lax.fori_loop(0, _NCH // 2, pair_body, 0)
    drain(0, set_a)


_sc_call = functools.partial(
    pl.kernel,
    out_type=jax.ShapeDtypeStruct((BATCH_K // _SR, _SR, NW_K), jnp.int32),
    mesh=plsc.VectorSubcoreMesh(core_axis_name="c", subcore_axis_name="s"),
    scratch_types=[
        pltpu.VMEM((_BPW,), jnp.int32),
        pltpu.VMEM((_BPW,), jnp.int32),
        pltpu.VMEM((_BPW,), jnp.int32),
        pltpu.VMEM((_SG, _SR, NW_K), jnp.int32),
        pltpu.VMEM((_SG, _SR, NW_K), jnp.int32),
        pltpu.VMEM((_SG, _SR, NW_K), jnp.int32),
        pltpu.VMEM((_SG, _SR, NW_K), jnp.int32),
        pltpu.VMEM((_SG, _SR, NW_K), jnp.int32),
        pltpu.VMEM((_SG, _SR, NW_K), jnp.int32),
        pltpu.SemaphoreType.DMA,
        pltpu.SemaphoreType.DMA,
    ],
)(_sc_body)


def kernel(bits, memory, positions):
    a0, a1, a2 = _addr_call(positions, bits)
    table = _tr_call(memory).reshape(N_HASH_K * RAM_K, NW_K)
    packed = _sc_call(table, a0, a1, a2)
    out32 = packed.reshape(BATCH_K, NW_K)
    return lax.bitcast_convert_type(out32, jnp.uint8).reshape(BATCH_K, N_BITS_K)


# speedup vs baseline: 1.7838x; 1.7838x over previous
import functools

import jax
import jax.numpy as jnp
from jax import lax
from jax.experimental import pallas as pl
from jax.experimental.pallas import tpu as pltpu
from jax.experimental.pallas import tpu_sc as plsc

N_BITS_K = 1024
HASH_BITS_K = 14
N_HASH_K = 3
BATCH_K = 16384
RAM_K = 2 ** HASH_BITS_K
NW_K = 1024 // 4


_BB = 512


def _addr_body(w_ref, bits_ref, o0, o1, o2):
    b = bits_ref[...].astype(jnp.float32)
    res = jnp.dot(b, w_ref[...], preferred_element_type=jnp.float32)
    resi = res.astype(jnp.int32)
    o0[...] = resi[:, 0:1]
    o1[...] = resi[:, 1:2] + RAM_K
    o2[...] = resi[:, 2:3] + 2 * RAM_K


def _addr_call(positions, bits):
    cols = (N_BITS_K - 1 - positions).reshape(-1)
    hh = jnp.repeat(jnp.arange(N_HASH_K), HASH_BITS_K)
    wts = jnp.tile(2.0 ** jnp.arange(HASH_BITS_K - 1, -1, -1), N_HASH_K)
    w = jnp.zeros((N_BITS_K, 8), jnp.float32).at[cols, hh].add(
        wts.astype(jnp.float32))
    grid = (BATCH_K // _BB,)
    out = jax.ShapeDtypeStruct((BATCH_K, 1), jnp.int32)
    a0, a1, a2 = pl.pallas_call(
        _addr_body,
        grid=grid,
        in_specs=[
            pl.BlockSpec((N_BITS_K, 8), lambda i: (0, 0)),
            pl.BlockSpec((_BB, N_BITS_K), lambda i: (i, 0)),
        ],
        out_specs=[
            pl.BlockSpec((_BB, 1), lambda i: (i, 0)),
            pl.BlockSpec((_BB, 1), lambda i: (i, 0)),
            pl.BlockSpec((_BB, 1), lambda i: (i, 0)),
        ],
        out_shape=[out, out, out],
    )(w, bits)
    return a0.reshape(-1), a1.reshape(-1), a2.reshape(-1)


_BA = 512
PW_K = 1024 // 32
_RW = 128


def _tr_body(s_ref, mem_ref, out_ref):
    x = mem_ref[0]
    t = lax.dot_general(
        x, s_ref[...],
        dimension_numbers=(((0,), (0,)), ((), ())),
        preferred_element_type=jnp.float32)
    acc = t[:, 0:PW_K].astype(jnp.int32)
    acc = acc + (t[:, PW_K:2 * PW_K].astype(jnp.int32) << 8)
    acc = acc + (t[:, 4 * PW_K:5 * PW_K].astype(jnp.int32) << 16)
    acc = acc + (t[:, 5 * PW_K:6 * PW_K].astype(jnp.int32) << 24)
    out_ref[0, :, 0:PW_K] = acc


def _pack_mat():
    n = jnp.arange(N_BITS_K)
    wq = n // 32
    k = n % 32
    g = k // 8
    col = jnp.where(g < 2, g * PW_K, (g + 2) * PW_K) + wq
    val = 2.0 ** (k % 8)
    s = jnp.zeros((N_BITS_K, 8 * PW_K), jnp.float32)
    return s.at[n, col].set(val.astype(jnp.float32))


def _tr_call(memory):
    grid = (N_HASH_K, RAM_K // _BA)
    smat = _pack_mat()
    return pl.pallas_call(
        _tr_body,
        grid=grid,
        in_specs=[pl.BlockSpec((N_BITS_K, 8 * PW_K), lambda h, ab: (0, 0)),
                  pl.BlockSpec((1, N_BITS_K, _BA), lambda h, ab: (h, 0, ab))],
        out_specs=pl.BlockSpec((1, _BA, _RW), lambda h, ab: (h, ab, 0)),
        out_shape=jax.ShapeDtypeStruct((N_HASH_K, RAM_K, _RW), jnp.int32),
    )(smat, memory)


_NSC = 32
_BPW = BATCH_K // _NSC
_CH = 64
_NCH = _BPW // _CH
_SG = 4
_SR = _CH // _SG


def _sc_body(table_hbm, a0_hbm, a1_hbm, a2_hbm, out_hbm,
             i0, i1, i2, p00, p01, p02, p10, p11, p12, ob0, ob1, sem_a, sem_b):
    wid = lax.axis_index("s") * 2 + lax.axis_index("c")
    base = wid * _BPW
    pltpu.sync_copy(a0_hbm.at[pl.ds(base, _BPW)], i0)
    pltpu.sync_copy(a1_hbm.at[pl.ds(base, _BPW)], i1)
    pltpu.sync_copy(a2_hbm.at[pl.ds(base, _BPW)], i2)

    set_a = (p00, p01, p02, ob0, sem_a)
    set_b = (p10, p11, p12, ob1, sem_b)

    lanes = lax.broadcasted_iota(jnp.int32, (16,), 0)
    shifts = 4 * (lanes & 7)

    def fire(c, bset):
        pb0, pb1, pb2, _, sm = bset
        sl = pl.ds(c * _CH, _CH)
        pltpu.async_copy(table_hbm.at[i0.at[sl]], pb0.reshape(_CH, _RW), sm)
        pltpu.async_copy(table_hbm.at[i1.at[sl]], pb1.reshape(_CH, _RW), sm)
        pltpu.async_copy(table_hbm.at[i2.at[sl]], pb2.reshape(_CH, _RW), sm)

    def drain(c, bset):
        pb0, pb1, pb2, _, sm = bset
        sl = pl.ds(c * _CH, _CH)
        pltpu.make_async_copy(table_hbm.at[i0.at[sl]], pb0.reshape(_CH, _RW), sm).wait()
        pltpu.make_async_copy(table_hbm.at[i1.at[sl]], pb1.reshape(_CH, _RW), sm).wait()
        pltpu.make_async_copy(table_hbm.at[i2.at[sl]], pb2.reshape(_CH, _RW), sm).wait()

    def compute_write(c, bset):
        pb0, pb1, pb2, ob, _ = bset

        def maj_body(s, _):
            for r in range(_SR):
                for jj in range(PW_K // 16):
                    sl = pl.ds(jj * 16, 16)
                    v0 = pb0[s, r, sl]
                    v1 = pb1[s, r, sl]
                    v2 = pb2[s, r, sl]
                    pb0[s, r, sl] = (v0 & v1) | (v2 & (v0 | v1))
            return 0

        lax.fori_loop(0, _SG, maj_body, 0)

        def exp_body(s, _):
            for r in range(_SR):
                va = pb0[s, r, pl.ds(0, 16)]
                vb = pb0[s, r, pl.ds(16, 16)]
                for j in range(N_BITS_K // 4 // 16):
                    src = va if j < 8 else vb
                    pidx = 2 * (j % 8) + (lanes >> 3)
                    pv = lax.gather(
                        src, pidx[:, None],
                        lax.GatherDimensionNumbers(
                            offset_dims=(), collapsed_slice_dims=(0,),
                            start_index_map=(0,)),
                        slice_sizes=(1,),
                        mode=lax.GatherScatterMode.PROMISE_IN_BOUNDS)
                    nib = (pv >> shifts) & 0xF
                    ob[s, r, pl.ds(j * 16, 16)] = (nib * 0x00204081) & 0x01010101
            return 0

        lax.fori_loop(0, _SG, exp_body, 0)
        pltpu.sync_copy(
            ob.reshape(_CH, NW_K),
            out_hbm.bitcast(jnp.int32).at[
                pl.ds(pl.multiple_of(base + c * _CH, 16), _CH)])

    fire(0, set_a)

    def pair_body(t, _):
        c0 = 2 * t
        c1 = 2 * t + 1
        fire(c1, set_b)
        drain(c0, set_a)
        compute_write(c0, set_a)
        fire((c1 + 1) % _NCH, set_a)
        drain(c1, set_b)
        compute_write(c1, set_b)
        return 0

    lax.fori_loop(0, _NCH // 2, pair_body, 0)
    drain(0, set_a)


_sc_call = functools.partial(
    pl.kernel,
    out_type=jax.ShapeDtypeStruct((4 * BATCH_K, NW_K), jnp.uint8),
    mesh=plsc.VectorSubcoreMesh(core_axis_name="c", subcore_axis_name="s"),
    scratch_types=[
        pltpu.VMEM((_BPW,), jnp.int32),
        pltpu.VMEM((_BPW,), jnp.int32),
        pltpu.VMEM((_BPW,), jnp.int32),
        pltpu.VMEM((_SG, _SR, _RW), jnp.int32),
        pltpu.VMEM((_SG, _SR, _RW), jnp.int32),
        pltpu.VMEM((_SG, _SR, _RW), jnp.int32),
        pltpu.VMEM((_SG, _SR, _RW), jnp.int32),
        pltpu.VMEM((_SG, _SR, _RW), jnp.int32),
        pltpu.VMEM((_SG, _SR, _RW), jnp.int32),
        pltpu.VMEM((_SG, _SR, NW_K), jnp.int32),
        pltpu.VMEM((_SG, _SR, NW_K), jnp.int32),
        pltpu.SemaphoreType.DMA,
        pltpu.SemaphoreType.DMA,
    ],
)(_sc_body)


def kernel(bits, memory, positions):
    a0, a1, a2 = _addr_call(positions, bits)
    table = _tr_call(memory).reshape(N_HASH_K * RAM_K, _RW)
    u8v = _sc_call(table, a0, a1, a2)
    return u8v.reshape(BATCH_K, 4, NW_K).transpose(0, 2, 1).reshape(
        BATCH_K, N_BITS_K)


# speedup vs baseline: 2.4601x; 1.3791x over previous
import functools

import jax
import jax.numpy as jnp
from jax import lax
from jax.experimental import pallas as pl
from jax.experimental.pallas import tpu as pltpu
from jax.experimental.pallas import tpu_sc as plsc

N_BITS_K = 1024
HASH_BITS_K = 14
N_HASH_K = 3
BATCH_K = 16384
RAM_K = 2 ** HASH_BITS_K
NW_K = 1024 // 4


_BB = 512


def _addr_body(w_ref, bits_ref, o0, o1, o2):
    b = bits_ref[...].astype(jnp.float32)
    res = jnp.dot(b, w_ref[...], preferred_element_type=jnp.float32)
    resi = res.astype(jnp.int32)
    o0[...] = resi[:, 0:1]
    o1[...] = resi[:, 1:2] + RAM_K
    o2[...] = resi[:, 2:3] + 2 * RAM_K


def _addr_call(positions, bits):
    cols = (N_BITS_K - 1 - positions).reshape(-1)
    hh = jnp.repeat(jnp.arange(N_HASH_K), HASH_BITS_K)
    wts = jnp.tile(2.0 ** jnp.arange(HASH_BITS_K - 1, -1, -1), N_HASH_K)
    w = jnp.zeros((N_BITS_K, 8), jnp.float32).at[cols, hh].add(
        wts.astype(jnp.float32))
    grid = (BATCH_K // _BB,)
    out = jax.ShapeDtypeStruct((BATCH_K, 1), jnp.int32)
    a0, a1, a2 = pl.pallas_call(
        _addr_body,
        grid=grid,
        in_specs=[
            pl.BlockSpec((N_BITS_K, 8), lambda i: (0, 0)),
            pl.BlockSpec((_BB, N_BITS_K), lambda i: (i, 0)),
        ],
        out_specs=[
            pl.BlockSpec((_BB, 1), lambda i: (i, 0)),
            pl.BlockSpec((_BB, 1), lambda i: (i, 0)),
            pl.BlockSpec((_BB, 1), lambda i: (i, 0)),
        ],
        out_shape=[out, out, out],
    )(w, bits)
    return a0.reshape(-1), a1.reshape(-1), a2.reshape(-1)


_BA = 512
PW_K = 1024 // 32
_RW = 128


def _tr_body(s_ref, mem_ref, out_ref):
    x = mem_ref[0]
    t = lax.dot_general(
        x, s_ref[...],
        dimension_numbers=(((0,), (0,)), ((), ())),
        preferred_element_type=jnp.float32)
    acc = t[:, 0:PW_K].astype(jnp.int32)
    acc = acc + (t[:, PW_K:2 * PW_K].astype(jnp.int32) << 8)
    acc = acc + (t[:, 4 * PW_K:5 * PW_K].astype(jnp.int32) << 16)
    acc = acc + (t[:, 5 * PW_K:6 * PW_K].astype(jnp.int32) << 24)
    out_ref[0, :, 0:PW_K] = acc


def _pack_mat():
    n = jnp.arange(N_BITS_K)
    wq = n // 32
    k = n % 32
    g = k // 8
    col = jnp.where(g < 2, g * PW_K, (g + 2) * PW_K) + wq
    val = 2.0 ** (k % 8)
    s = jnp.zeros((N_BITS_K, 8 * PW_K), jnp.float32)
    return s.at[n, col].set(val.astype(jnp.float32))


def _tr_call(memory):
    grid = (N_HASH_K, RAM_K // _BA)
    smat = _pack_mat()
    return pl.pallas_call(
        _tr_body,
        grid=grid,
        in_specs=[pl.BlockSpec((N_BITS_K, 8 * PW_K), lambda h, ab: (0, 0)),
                  pl.BlockSpec((1, N_BITS_K, _BA), lambda h, ab: (h, 0, ab))],
        out_specs=pl.BlockSpec((1, _BA, _RW), lambda h, ab: (h, ab, 0)),
        out_shape=jax.ShapeDtypeStruct((N_HASH_K, RAM_K, _RW), jnp.int32),
    )(smat, memory)


_NSC = 32
_BPW = BATCH_K // _NSC
_CH = 64
_NCH = _BPW // _CH
_SG = _CH // 4


def _sc_body(table_hbm, a0_hbm, a1_hbm, a2_hbm, out_hbm,
             i0, i1, i2, p00, p01, p02, p10, p11, p12, ob0, ob1, sem_a, sem_b):
    wid = lax.axis_index("s") * 2 + lax.axis_index("c")
    base = wid * _BPW
    pltpu.sync_copy(a0_hbm.at[pl.ds(base, _BPW)], i0)
    pltpu.sync_copy(a1_hbm.at[pl.ds(base, _BPW)], i1)
    pltpu.sync_copy(a2_hbm.at[pl.ds(base, _BPW)], i2)

    set_a = (p00, p01, p02, ob0, sem_a)
    set_b = (p10, p11, p12, ob1, sem_b)

    lanes = lax.broadcasted_iota(jnp.int32, (16,), 0)

    def _splat(v, lane):
        return lax.gather(
            v, (lanes * 0 + lane)[:, None],
            lax.GatherDimensionNumbers(
                offset_dims=(), collapsed_slice_dims=(0,),
                start_index_map=(0,)),
            slice_sizes=(1,),
            mode=lax.GatherScatterMode.PROMISE_IN_BOUNDS)

    def fire(c, bset):
        pb0, pb1, pb2, _, sm = bset
        sl = pl.ds(c * _CH, _CH)
        pltpu.async_copy(table_hbm.at[i0.at[sl]], pb0.reshape(_CH, _RW), sm)
        pltpu.async_copy(table_hbm.at[i1.at[sl]], pb1.reshape(_CH, _RW), sm)
        pltpu.async_copy(table_hbm.at[i2.at[sl]], pb2.reshape(_CH, _RW), sm)

    def drain(c, bset):
        pb0, pb1, pb2, _, sm = bset
        sl = pl.ds(c * _CH, _CH)
        pltpu.make_async_copy(table_hbm.at[i0.at[sl]], pb0.reshape(_CH, _RW), sm).wait()
        pltpu.make_async_copy(table_hbm.at[i1.at[sl]], pb1.reshape(_CH, _RW), sm).wait()
        pltpu.make_async_copy(table_hbm.at[i2.at[sl]], pb2.reshape(_CH, _RW), sm).wait()

    def compute_write(c, bset):
        pb0, pb1, pb2, ob, _ = bset

        def maj_body(t, _):
            for q in range(4):
                for jj in range(PW_K // 16):
                    sl = pl.ds(jj * 16, 16)
                    v0 = pb0[t, q, sl]
                    v1 = pb1[t, q, sl]
                    v2 = pb2[t, q, sl]
                    pb0[t, q, sl] = (v0 & v1) | (v2 & (v0 | v1))
            return 0

        lax.fori_loop(0, _SG, maj_body, 0)

        def exp_body(t, _):
            v = [[pb0[t, q, pl.ds(g * 16, 16)] for g in range(2)]
                 for q in range(4)]
            for wsrc in range(PW_K):
                g, lane = wsrc // 16, wsrc % 16
                sp = [_splat(v[q][g], lane) for q in range(4)]
                for half in range(2):
                    sh = 16 * half + lanes
                    acc = (sp[0] >> sh) & 1
                    acc = acc | (((sp[1] >> sh) & 1) << 8)
                    acc = acc | (((sp[2] >> sh) & 1) << 16)
                    acc = acc | (((sp[3] >> sh) & 1) << 24)
                    ob[t, 0, pl.ds((2 * wsrc + half) * 16, 16)] = acc
            return 0

        lax.fori_loop(0, _SG, exp_body, 0)
        pltpu.sync_copy(
            ob.reshape(_SG, N_BITS_K),
            out_hbm.bitcast(jnp.int32).at[
                pl.ds(pl.multiple_of((base + c * _CH) // 4, 16), _SG)])

    fire(0, set_a)

    def pair_body(t, _):
        c0 = 2 * t
        c1 = 2 * t + 1
        fire(c1, set_b)
        drain(c0, set_a)
        compute_write(c0, set_a)
        fire((c1 + 1) % _NCH, set_a)
        drain(c1, set_b)
        compute_write(c1, set_b)
        return 0

    lax.fori_loop(0, _NCH // 2, pair_body, 0)
    drain(0, set_a)


_sc_call = functools.partial(
    pl.kernel,
    out_type=jax.ShapeDtypeStruct((BATCH_K, N_BITS_K), jnp.uint8),
    mesh=plsc.VectorSubcoreMesh(core_axis_name="c", subcore_axis_name="s"),
    scratch_types=[
        pltpu.VMEM((_BPW,), jnp.int32),
        pltpu.VMEM((_BPW,), jnp.int32),
        pltpu.VMEM((_BPW,), jnp.int32),
        pltpu.VMEM((_SG, 4, _RW), jnp.int32),
        pltpu.VMEM((_SG, 4, _RW), jnp.int32),
        pltpu.VMEM((_SG, 4, _RW), jnp.int32),
        pltpu.VMEM((_SG, 4, _RW), jnp.int32),
        pltpu.VMEM((_SG, 4, _RW), jnp.int32),
        pltpu.VMEM((_SG, 4, _RW), jnp.int32),
        pltpu.VMEM((_SG, 1, N_BITS_K), jnp.int32),
        pltpu.VMEM((_SG, 1, N_BITS_K), jnp.int32),
        pltpu.SemaphoreType.DMA,
        pltpu.SemaphoreType.DMA,
    ],
)(_sc_body)


def kernel(bits, memory, positions):
    a0, a1, a2 = _addr_call(positions, bits)
    table = _tr_call(memory).reshape(N_HASH_K * RAM_K, _RW)
    return _sc_call(table, a0, a1, a2)


# speedup vs baseline: 2.9622x; 1.2041x over previous
import functools

import jax
import jax.numpy as jnp
from jax import lax
from jax.experimental import pallas as pl
from jax.experimental.pallas import tpu as pltpu
from jax.experimental.pallas import tpu_sc as plsc

N_BITS_K = 1024
HASH_BITS_K = 14
N_HASH_K = 3
BATCH_K = 16384
RAM_K = 2 ** HASH_BITS_K
NW_K = 1024 // 4


_BB = 512


def _addr_body(w_ref, bits_ref, o0, o1, o2):
    b = bits_ref[...].astype(jnp.float32)
    res = jnp.dot(b, w_ref[...], preferred_element_type=jnp.float32)
    resi = res.astype(jnp.int32)
    o0[...] = resi[:, 0:1].reshape(_BB)
    o1[...] = (resi[:, 1:2] + RAM_K).reshape(_BB)
    o2[...] = (resi[:, 2:3] + 2 * RAM_K).reshape(_BB)


def _addr_call(positions, bits):
    cols = (N_BITS_K - 1 - positions).reshape(-1)
    hh = jnp.repeat(jnp.arange(N_HASH_K), HASH_BITS_K)
    wts = jnp.tile(2.0 ** jnp.arange(HASH_BITS_K - 1, -1, -1), N_HASH_K)
    w = jnp.zeros((N_BITS_K, 8), jnp.float32).at[cols, hh].add(
        wts.astype(jnp.float32))
    grid = (BATCH_K // _BB,)
    out = jax.ShapeDtypeStruct((BATCH_K,), jnp.int32)
    a0, a1, a2 = pl.pallas_call(
        _addr_body,
        grid=grid,
        in_specs=[
            pl.BlockSpec((N_BITS_K, 8), lambda i: (0, 0)),
            pl.BlockSpec((_BB, N_BITS_K), lambda i: (i, 0)),
        ],
        out_specs=[
            pl.BlockSpec((_BB,), lambda i: (i,)),
            pl.BlockSpec((_BB,), lambda i: (i,)),
            pl.BlockSpec((_BB,), lambda i: (i,)),
        ],
        out_shape=[out, out, out],
    )(w, bits)
    return a0, a1, a2


_BA = 1024
PW_K = 1024 // 32
_RW = 128


def _tr_body(s_ref, mem_ref, out_ref):
    x = mem_ref[0]
    t = lax.dot_general(
        x, s_ref[...],
        dimension_numbers=(((0,), (0,)), ((), ())),
        preferred_element_type=jnp.float32)
    acc = t[:, 0:PW_K].astype(jnp.int32)
    acc = acc + (t[:, PW_K:2 * PW_K].astype(jnp.int32) << 8)
    acc = acc + (t[:, 4 * PW_K:5 * PW_K].astype(jnp.int32) << 16)
    acc = acc + (t[:, 5 * PW_K:6 * PW_K].astype(jnp.int32) << 24)
    out_ref[0, :, 0:PW_K] = acc


def _pack_mat():
    n = jnp.arange(N_BITS_K)
    wq = n // 32
    k = n % 32
    g = k // 8
    col = jnp.where(g < 2, g * PW_K, (g + 2) * PW_K) + wq
    val = 2.0 ** (k % 8)
    s = jnp.zeros((N_BITS_K, 8 * PW_K), jnp.float32)
    return s.at[n, col].set(val.astype(jnp.float32))


def _tr_call(memory):
    grid = (N_HASH_K, RAM_K // _BA)
    smat = _pack_mat()
    return pl.pallas_call(
        _tr_body,
        grid=grid,
        in_specs=[pl.BlockSpec((N_BITS_K, 8 * PW_K), lambda h, ab: (0, 0)),
                  pl.BlockSpec((1, N_BITS_K, _BA), lambda h, ab: (h, 0, ab))],
        out_specs=pl.BlockSpec((1, _BA, _RW), lambda h, ab: (h, ab, 0)),
        out_shape=jax.ShapeDtypeStruct((N_HASH_K, RAM_K, _RW), jnp.int32),
    )(smat, memory)


_NSC = 32
_BPW = BATCH_K // _NSC
_CH = 64
_NCH = _BPW // _CH
_SG = _CH // 4


def _sc_body(table_hbm, a0_hbm, a1_hbm, a2_hbm, out_hbm,
             i0, i1, i2, p00, p01, p02, p10, p11, p12, ob0, ob1, sem_a, sem_b):
    wid = lax.axis_index("s") * 2 + lax.axis_index("c")
    base = wid * _BPW
    pltpu.sync_copy(a0_hbm.at[pl.ds(base, _BPW)], i0)
    pltpu.sync_copy(a1_hbm.at[pl.ds(base, _BPW)], i1)
    pltpu.sync_copy(a2_hbm.at[pl.ds(base, _BPW)], i2)

    set_a = (p00, p01, p02, ob0, sem_a)
    set_b = (p10, p11, p12, ob1, sem_b)

    lanes = lax.broadcasted_iota(jnp.int32, (16,), 0)

    def _splat(v, lane):
        return lax.gather(
            v, (lanes * 0 + lane)[:, None],
            lax.GatherDimensionNumbers(
                offset_dims=(), collapsed_slice_dims=(0,),
                start_index_map=(0,)),
            slice_sizes=(1,),
            mode=lax.GatherScatterMode.PROMISE_IN_BOUNDS)

    def fire(c, bset):
        pb0, pb1, pb2, _, sm = bset
        sl = pl.ds(c * _CH, _CH)
        pltpu.async_copy(table_hbm.at[i0.at[sl]], pb0.reshape(_CH, _RW), sm)
        pltpu.async_copy(table_hbm.at[i1.at[sl]], pb1.reshape(_CH, _RW), sm)
        pltpu.async_copy(table_hbm.at[i2.at[sl]], pb2.reshape(_CH, _RW), sm)

    def drain(c, bset):
        pb0, pb1, pb2, _, sm = bset
        sl = pl.ds(c * _CH, _CH)
        pltpu.make_async_copy(table_hbm.at[i0.at[sl]], pb0.reshape(_CH, _RW), sm).wait()
        pltpu.make_async_copy(table_hbm.at[i1.at[sl]], pb1.reshape(_CH, _RW), sm).wait()
        pltpu.make_async_copy(table_hbm.at[i2.at[sl]], pb2.reshape(_CH, _RW), sm).wait()

    def compute_write(c, bset):
        pb0, pb1, pb2, ob, _ = bset

        def maj_body(t, _):
            for q in range(4):
                for jj in range(PW_K // 16):
                    sl = pl.ds(jj * 16, 16)
                    v0 = pb0[t, q, sl]
                    v1 = pb1[t, q, sl]
                    v2 = pb2[t, q, sl]
                    pb0[t, q, sl] = (v0 & v1) | (v2 & (v0 | v1))
            return 0

        lax.fori_loop(0, _SG, maj_body, 0)

        def exp_body(t, _):
            v = [[pb0[t, q, pl.ds(g * 16, 16)] for g in range(2)]
                 for q in range(4)]
            for wsrc in range(PW_K):
                g, lane = wsrc // 16, wsrc % 16
                sp = [_splat(v[q][g], lane) for q in range(4)]
                for half in range(2):
                    sh = 16 * half + lanes
                    acc = (sp[0] >> sh) & 1
                    acc = acc | (((sp[1] >> sh) & 1) << 8)
                    acc = acc | (((sp[2] >> sh) & 1) << 16)
                    acc = acc | (((sp[3] >> sh) & 1) << 24)
                    ob[t, 0, pl.ds((2 * wsrc + half) * 16, 16)] = acc
            return 0

        lax.fori_loop(0, _SG, exp_body, 0)
        pltpu.sync_copy(
            ob.reshape(_SG, N_BITS_K),
            out_hbm.bitcast(jnp.int32).at[
                pl.ds(pl.multiple_of((base + c * _CH) // 4, 16), _SG)])

    fire(0, set_a)

    def pair_body(t, _):
        c0 = 2 * t
        c1 = 2 * t + 1
        fire(c1, set_b)
        drain(c0, set_a)
        compute_write(c0, set_a)
        fire((c1 + 1) % _NCH, set_a)
        drain(c1, set_b)
        compute_write(c1, set_b)
        return 0

    lax.fori_loop(0, _NCH // 2, pair_body, 0)
    drain(0, set_a)


_sc_call = functools.partial(
    pl.kernel,
    out_type=jax.ShapeDtypeStruct((BATCH_K, N_BITS_K), jnp.uint8),
    mesh=plsc.VectorSubcoreMesh(core_axis_name="c", subcore_axis_name="s"),
    scratch_types=[
        pltpu.VMEM((_BPW,), jnp.int32),
        pltpu.VMEM((_BPW,), jnp.int32),
        pltpu.VMEM((_BPW,), jnp.int32),
        pltpu.VMEM((_SG, 4, _RW), jnp.int32),
        pltpu.VMEM((_SG, 4, _RW), jnp.int32),
        pltpu.VMEM((_SG, 4, _RW), jnp.int32),
        pltpu.VMEM((_SG, 4, _RW), jnp.int32),
        pltpu.VMEM((_SG, 4, _RW), jnp.int32),
        pltpu.VMEM((_SG, 4, _RW), jnp.int32),
        pltpu.VMEM((_SG, 1, N_BITS_K), jnp.int32),
        pltpu.VMEM((_SG, 1, N_BITS_K), jnp.int32),
        pltpu.SemaphoreType.DMA,
        pltpu.SemaphoreType.DMA,
    ],
)(_sc_body)


def kernel(bits, memory, positions):
    a0, a1, a2 = _addr_call(positions, bits)
    table = _tr_call(memory).reshape(N_HASH_K * RAM_K, _RW)
    return _sc_call(table, a0, a1, a2)


# speedup vs baseline: 3.2937x; 1.1119x over previous
import functools

import jax
import jax.numpy as jnp
from jax import lax
from jax.experimental import pallas as pl
from jax.experimental.pallas import tpu as pltpu
from jax.experimental.pallas import tpu_sc as plsc

N_BITS_K = 1024
HASH_BITS_K = 14
N_HASH_K = 3
BATCH_K = 16384
RAM_K = 2 ** HASH_BITS_K
NW_K = 1024 // 4


_BB = 1024


def _addr_body(w_ref, bits_ref, o0, o1, o2):
    b = bits_ref[...].astype(jnp.float32)
    res = jnp.dot(b, w_ref[...], preferred_element_type=jnp.float32)
    resi = res.astype(jnp.int32)
    o0[...] = resi[:, 0:1].reshape(_BB)
    o1[...] = (resi[:, 1:2] + RAM_K).reshape(_BB)
    o2[...] = (resi[:, 2:3] + 2 * RAM_K).reshape(_BB)


def _addr_call(positions, bits):
    cols = (N_BITS_K - 1 - positions).reshape(-1)
    hh = jnp.repeat(jnp.arange(N_HASH_K), HASH_BITS_K)
    wts = jnp.tile(2.0 ** jnp.arange(HASH_BITS_K - 1, -1, -1), N_HASH_K)
    w = jnp.zeros((N_BITS_K, 8), jnp.float32).at[cols, hh].add(
        wts.astype(jnp.float32))
    grid = (BATCH_K // _BB,)
    out = jax.ShapeDtypeStruct((BATCH_K,), jnp.int32)
    a0, a1, a2 = pl.pallas_call(
        _addr_body,
        grid=grid,
        in_specs=[
            pl.BlockSpec((N_BITS_K, 8), lambda i: (0, 0)),
            pl.BlockSpec((_BB, N_BITS_K), lambda i: (i, 0)),
        ],
        out_specs=[
            pl.BlockSpec((_BB,), lambda i: (i,)),
            pl.BlockSpec((_BB,), lambda i: (i,)),
            pl.BlockSpec((_BB,), lambda i: (i,)),
        ],
        out_shape=[out, out, out],
    )(w, bits)
    return a0, a1, a2


_BA = 2048
PW_K = 1024 // 32
_RW = 128


def _tr_body(s_ref, mem_ref, out_ref):
    x = mem_ref[0]
    t = lax.dot_general(
        x, s_ref[...],
        dimension_numbers=(((0,), (0,)), ((), ())),
        preferred_element_type=jnp.float32)
    acc = t[:, 0:PW_K].astype(jnp.int32)
    acc = acc + (t[:, PW_K:2 * PW_K].astype(jnp.int32) << 8)
    acc = acc + (t[:, 4 * PW_K:5 * PW_K].astype(jnp.int32) << 16)
    acc = acc + (t[:, 5 * PW_K:6 * PW_K].astype(jnp.int32) << 24)
    out_ref[0, :, 0:PW_K] = acc


def _pack_mat():
    n = jnp.arange(N_BITS_K)
    wq = n // 32
    k = n % 32
    g = k // 8
    col = jnp.where(g < 2, g * PW_K, (g + 2) * PW_K) + wq
    val = 2.0 ** (k % 8)
    s = jnp.zeros((N_BITS_K, 8 * PW_K), jnp.float32)
    return s.at[n, col].set(val.astype(jnp.float32))


def _tr_call(memory):
    grid = (N_HASH_K, RAM_K // _BA)
    smat = _pack_mat()
    return pl.pallas_call(
        _tr_body,
        grid=grid,
        in_specs=[pl.BlockSpec((N_BITS_K, 8 * PW_K), lambda h, ab: (0, 0)),
                  pl.BlockSpec((1, N_BITS_K, _BA), lambda h, ab: (h, 0, ab))],
        out_specs=pl.BlockSpec((1, _BA, _RW), lambda h, ab: (h, ab, 0)),
        out_shape=jax.ShapeDtypeStruct((N_HASH_K, RAM_K, _RW), jnp.int32),
    )(smat, memory)


_NSC = 32
_BPW = BATCH_K // _NSC
_CH = 64
_NCH = _BPW // _CH
_SG = _CH // 4


def _sc_body(table_hbm, a0_hbm, a1_hbm, a2_hbm, out_hbm,
             i0, i1, i2, p00, p01, p02, p10, p11, p12, ob0, ob1, sem_a, sem_b):
    wid = lax.axis_index("s") * 2 + lax.axis_index("c")
    base = wid * _BPW
    pltpu.sync_copy(a0_hbm.at[pl.ds(base, _BPW)], i0)
    pltpu.sync_copy(a1_hbm.at[pl.ds(base, _BPW)], i1)
    pltpu.sync_copy(a2_hbm.at[pl.ds(base, _BPW)], i2)

    set_a = (p00, p01, p02, ob0, sem_a)
    set_b = (p10, p11, p12, ob1, sem_b)

    lanes = lax.broadcasted_iota(jnp.int32, (16,), 0)

    def _splat(v, lane):
        return lax.gather(
            v, (lanes * 0 + lane)[:, None],
            lax.GatherDimensionNumbers(
                offset_dims=(), collapsed_slice_dims=(0,),
                start_index_map=(0,)),
            slice_sizes=(1,),
            mode=lax.GatherScatterMode.PROMISE_IN_BOUNDS)

    def fire(c, bset):
        pb0, pb1, pb2, _, sm = bset
        sl = pl.ds(c * _CH, _CH)
        pltpu.async_copy(table_hbm.at[i0.at[sl]], pb0.reshape(_CH, _RW), sm)
        pltpu.async_copy(table_hbm.at[i1.at[sl]], pb1.reshape(_CH, _RW), sm)
        pltpu.async_copy(table_hbm.at[i2.at[sl]], pb2.reshape(_CH, _RW), sm)

    def drain(c, bset):
        pb0, pb1, pb2, _, sm = bset
        sl = pl.ds(c * _CH, _CH)
        pltpu.make_async_copy(table_hbm.at[i0.at[sl]], pb0.reshape(_CH, _RW), sm).wait()
        pltpu.make_async_copy(table_hbm.at[i1.at[sl]], pb1.reshape(_CH, _RW), sm).wait()
        pltpu.make_async_copy(table_hbm.at[i2.at[sl]], pb2.reshape(_CH, _RW), sm).wait()

    def compute_write(c, bset):
        pb0, pb1, pb2, ob, _ = bset

        def maj_body(t, _):
            for q in range(4):
                for jj in range(PW_K // 16):
                    sl = pl.ds(jj * 16, 16)
                    v0 = pb0[t, q, sl]
                    v1 = pb1[t, q, sl]
                    v2 = pb2[t, q, sl]
                    pb0[t, q, sl] = (v0 & v1) | (v2 & (v0 | v1))
            return 0

        lax.fori_loop(0, _SG, maj_body, 0)

        def exp_body(t, _):
            v = [[pb0[t, q, pl.ds(g * 16, 16)] for g in range(2)]
                 for q in range(4)]
            for wsrc in range(PW_K):
                g, lane = wsrc // 16, wsrc % 16
                sp = [_splat(v[q][g], lane) for q in range(4)]
                for half in range(2):
                    sh = 16 * half + lanes
                    acc = (sp[0] >> sh) & 1
                    acc = acc | (((sp[1] >> sh) & 1) << 8)
                    acc = acc | (((sp[2] >> sh) & 1) << 16)
                    acc = acc | (((sp[3] >> sh) & 1) << 24)
                    ob[t, 0, pl.ds((2 * wsrc + half) * 16, 16)] = acc
            return 0

        lax.fori_loop(0, _SG, exp_body, 0)
        pltpu.sync_copy(
            ob.reshape(_SG, N_BITS_K),
            out_hbm.bitcast(jnp.int32).at[
                pl.ds(pl.multiple_of((base + c * _CH) // 4, 16), _SG)])

    fire(0, set_a)

    def pair_body(t, _):
        c0 = 2 * t
        c1 = 2 * t + 1
        fire(c1, set_b)
        drain(c0, set_a)
        compute_write(c0, set_a)
        fire((c1 + 1) % _NCH, set_a)
        drain(c1, set_b)
        compute_write(c1, set_b)
        return 0

    lax.fori_loop(0, _NCH // 2, pair_body, 0)
    drain(0, set_a)


_sc_call = functools.partial(
    pl.kernel,
    out_type=jax.ShapeDtypeStruct((BATCH_K, N_BITS_K), jnp.uint8),
    mesh=plsc.VectorSubcoreMesh(core_axis_name="c", subcore_axis_name="s"),
    scratch_types=[
        pltpu.VMEM((_BPW,), jnp.int32),
        pltpu.VMEM((_BPW,), jnp.int32),
        pltpu.VMEM((_BPW,), jnp.int32),
        pltpu.VMEM((_SG, 4, _RW), jnp.int32),
        pltpu.VMEM((_SG, 4, _RW), jnp.int32),
        pltpu.VMEM((_SG, 4, _RW), jnp.int32),
        pltpu.VMEM((_SG, 4, _RW), jnp.int32),
        pltpu.VMEM((_SG, 4, _RW), jnp.int32),
        pltpu.VMEM((_SG, 4, _RW), jnp.int32),
        pltpu.VMEM((_SG, 1, N_BITS_K), jnp.int32),
        pltpu.VMEM((_SG, 1, N_BITS_K), jnp.int32),
        pltpu.SemaphoreType.DMA,
        pltpu.SemaphoreType.DMA,
    ],
)(_sc_body)


def kernel(bits, memory, positions):
    a0, a1, a2 = _addr_call(positions, bits)
    table = _tr_call(memory).reshape(N_HASH_K * RAM_K, _RW)
    return _sc_call(table, a0, a1, a2)
